# baseline (speedup 1.0000x reference)
import functools

import jax
import jax.numpy as jnp
from jax import lax
from jax.experimental import pallas as pl
from jax.experimental.pallas import tpu as pltpu
from jax.experimental.pallas import tpu_sc as plsc

EPS = 0.001
R = 1024
DEPTH = 4


def _sc_gather(ids_hbm, mean_hbm, std_hbm, mean_out, scale_out,
               idx_v, mrows_v, srows_v, sem):
    B, D = 4, 1024
    wid = lax.axis_index("s") * 2 + lax.axis_index("c")

    @pl.when(wid == 0)
    def _():
        pltpu.sync_copy(ids_hbm, idx_v)
        pltpu.async_copy(mean_hbm.at[idx_v], mrows_v, sem).wait()
        pltpu.async_copy(std_hbm.at[idx_v], srows_v, sem).wait()
        for b in range(B):
            for j in range(D // 16):
                s = srows_v[b, pl.ds(j * 16, 16)]
                srows_v[b, pl.ds(j * 16, 16)] = 1.0 / (jnp.exp(s) + EPS)
        pltpu.sync_copy(mrows_v, mean_out)
        pltpu.sync_copy(srows_v, scale_out)


def _sc_lookup(ids, initial_mean, initial_std):
    B, D = 4, 1024
    mesh = plsc.VectorSubcoreMesh(core_axis_name="c", subcore_axis_name="s")
    f = pl.kernel(
        _sc_gather,
        out_type=[jax.ShapeDtypeStruct((B, D), jnp.float32),
                  jax.ShapeDtypeStruct((B, D), jnp.float32)],
        mesh=mesh,
        scratch_types=[
            pltpu.VMEM((B,), jnp.int32),
            pltpu.VMEM((B, D), jnp.float32),
            pltpu.VMEM((B, D), jnp.float32),
            pltpu.SemaphoreType.DMA,
        ],
    )
    return f(ids, initial_mean, initial_std)


def _tc_body(mean_ref, scale_ref, x_hbm, o_hbm, in_buf, out_buf, in_sems, out_sems):
    B = 4
    S_PER_B = 2048
    nchunks = (B * S_PER_B) // R
    chunks_per_b = S_PER_B // R

    for s in range(DEPTH):
        pltpu.make_async_copy(
            x_hbm.at[pl.ds(s * R, R), :], in_buf.at[s], in_sems.at[s]
        ).start()

    def outer(o, _):
        for s in range(DEPTH):
            c = o * DEPTH + s
            b = c // chunks_per_b
            pltpu.make_async_copy(
                x_hbm.at[pl.ds(c * R, R), :], in_buf.at[s], in_sems.at[s]
            ).wait()

            @pl.when(c >= DEPTH)
            def _():
                pltpu.make_async_copy(
                    out_buf.at[s], o_hbm.at[pl.ds((c - DEPTH) * R, R), :],
                    out_sems.at[s]
                ).wait()

            mrow = mean_ref[pl.ds(b, 1), :]
            srow = scale_ref[pl.ds(b, 1), :]
            out_buf[s] = (in_buf[s] - mrow) * srow

            pltpu.make_async_copy(
                out_buf.at[s], o_hbm.at[pl.ds(c * R, R), :], out_sems.at[s]
            ).start()

            @pl.when(c + DEPTH < nchunks)
            def _():
                pltpu.make_async_copy(
                    x_hbm.at[pl.ds((c + DEPTH) * R, R), :], in_buf.at[s],
                    in_sems.at[s]
                ).start()
        return ()

    lax.fori_loop(0, nchunks // DEPTH, outer, (), unroll=False)

    for s in range(DEPTH):
        c = nchunks - DEPTH + s
        pltpu.make_async_copy(
            out_buf.at[s], o_hbm.at[pl.ds(c * R, R), :], out_sems.at[s]
        ).wait()


def kernel(x, context_id, initial_mean, initial_std):
    B, S, D = x.shape
    ids = context_id.reshape(-1)
    mean_rows, scale_rows = _sc_lookup(ids, initial_mean, initial_std)
    x2 = x.reshape(B * S, D)
    out = pl.pallas_call(
        _tc_body,
        grid=(),
        in_specs=[
            pl.BlockSpec(memory_space=pltpu.VMEM),
            pl.BlockSpec(memory_space=pltpu.VMEM),
            pl.BlockSpec(memory_space=pl.ANY),
        ],
        out_specs=pl.BlockSpec(memory_space=pl.ANY),
        out_shape=jax.ShapeDtypeStruct((B * S, D), x.dtype),
        scratch_shapes=[
            pltpu.VMEM((DEPTH, R, D), jnp.float32),
            pltpu.VMEM((DEPTH, R, D), jnp.float32),
            pltpu.SemaphoreType.DMA((DEPTH,)),
            pltpu.SemaphoreType.DMA((DEPTH,)),
        ],
    )(mean_rows, scale_rows, x2)
    return out.reshape(B, S, D)


# speedup vs baseline: 2.3197x; 2.3197x over previous
import jax
import jax.numpy as jnp
from jax import lax
from jax.experimental import pallas as pl
from jax.experimental.pallas import tpu as pltpu

EPS = 0.001
R = 2048
DEPTH = 2


def _body(ids_ref, mean_ref, std_ref, x_hbm, o_hbm,
          in_buf, out_buf, scale_s, mean_s, in_sems, out_sems):
    B = 4
    S_PER_B = 2048
    nchunks = (B * S_PER_B) // R
    chunks_per_b = S_PER_B // R

    for b in range(B):
        idx = ids_ref[b]
        m = mean_ref[pl.ds(idx, 1), :]
        s = std_ref[pl.ds(idx, 1), :]
        mean_s[pl.ds(b, 1), :] = m
        scale_s[pl.ds(b, 1), :] = 1.0 / (jnp.exp(s) + EPS)

    for s in range(DEPTH):
        pltpu.make_async_copy(
            x_hbm.at[pl.ds(s * R, R), :], in_buf.at[s], in_sems.at[s]
        ).start()

    def outer(o, _):
        for s in range(DEPTH):
            c = o * DEPTH + s
            b = c // chunks_per_b
            pltpu.make_async_copy(
                x_hbm.at[pl.ds(c * R, R), :], in_buf.at[s], in_sems.at[s]
            ).wait()

            @pl.when(c >= DEPTH)
            def _():
                pltpu.make_async_copy(
                    out_buf.at[s], o_hbm.at[pl.ds((c - DEPTH) * R, R), :],
                    out_sems.at[s]
                ).wait()

            mrow = mean_s[pl.ds(b, 1), :]
            srow = scale_s[pl.ds(b, 1), :]
            out_buf[s] = (in_buf[s] - mrow) * srow

            pltpu.make_async_copy(
                out_buf.at[s], o_hbm.at[pl.ds(c * R, R), :], out_sems.at[s]
            ).start()

            @pl.when(c + DEPTH < nchunks)
            def _():
                pltpu.make_async_copy(
                    x_hbm.at[pl.ds((c + DEPTH) * R, R), :], in_buf.at[s],
                    in_sems.at[s]
                ).start()
        return ()

    lax.fori_loop(0, nchunks // DEPTH, outer, (), unroll=False)

    for s in range(DEPTH):
        c = nchunks - DEPTH + s
        pltpu.make_async_copy(
            out_buf.at[s], o_hbm.at[pl.ds(c * R, R), :], out_sems.at[s]
        ).wait()


def kernel(x, context_id, initial_mean, initial_std):
    B, S, D = x.shape
    ids = context_id.reshape(-1)
    x2 = x.reshape(B * S, D)
    out = pl.pallas_call(
        _body,
        grid=(),
        in_specs=[
            pl.BlockSpec(memory_space=pltpu.SMEM),
            pl.BlockSpec(memory_space=pltpu.VMEM),
            pl.BlockSpec(memory_space=pltpu.VMEM),
            pl.BlockSpec(memory_space=pl.ANY),
        ],
        out_specs=pl.BlockSpec(memory_space=pl.ANY),
        out_shape=jax.ShapeDtypeStruct((B * S, D), x.dtype),
        scratch_shapes=[
            pltpu.VMEM((DEPTH, R, D), jnp.float32),
            pltpu.VMEM((DEPTH, R, D), jnp.float32),
            pltpu.VMEM((B, D), jnp.float32),
            pltpu.VMEM((B, D), jnp.float32),
            pltpu.SemaphoreType.DMA((DEPTH,)),
            pltpu.SemaphoreType.DMA((DEPTH,)),
        ],
    )(ids, initial_mean, initial_std, x2)
    return out.reshape(B, S, D)
